# baseline (speedup 1.0000x reference)
import functools

import jax
import jax.numpy as jnp
from jax import lax
from jax.experimental import pallas as pl
from jax.experimental.pallas import tpu as pltpu
from jax.experimental.pallas import tpu_sc as plsc

EMBED = 64
PAIR = 2 * EMBED
LANES = 16
NCORES = 2
NSUB = 16
NWORKERS = NCORES * NSUB
IDX_CHUNK = 128
ROUND = 256
PAIR_BLOCK = 4000


def _pair_body(x_ref, o_ref):
    o_ref[:, 0:EMBED] = x_ref[pl.Slice(0, PAIR_BLOCK, 2), :]
    o_ref[:, EMBED:PAIR] = x_ref[pl.Slice(1, PAIR_BLOCK, 2), :]


def _tc_pair(table):
    vocab = table.shape[0]
    return pl.pallas_call(
        _pair_body,
        grid=(vocab // (2 * PAIR_BLOCK),),
        in_specs=[pl.BlockSpec((2 * PAIR_BLOCK, EMBED), lambda i: (i, 0))],
        out_specs=pl.BlockSpec((PAIR_BLOCK, PAIR), lambda i: (i, 0)),
        out_shape=jax.ShapeDtypeStruct((vocab // 2, PAIR), jnp.float32),
    )(table)


def _sc_body(bpw, tid_hbm, cid_hbm, table_in, table_out, o_hbm,
             ids_t, ids_c, pid_t, pid_c, rows_t, rows_c, tpose, out_v, sem):
    wid = lax.axis_index("s") * NCORES + lax.axis_index("c")
    base = wid * bpw

    pltpu.sync_copy(tid_hbm.at[pl.ds(base, bpw)], ids_t)
    pltpu.sync_copy(cid_hbm.at[pl.ds(base, bpw)], ids_c)

    for j in range(bpw // LANES):
        sl = pl.ds(j * LANES, LANES)
        pid_t[sl] = lax.shift_right_logical(ids_t[sl], 1)
        pid_c[sl] = lax.shift_right_logical(ids_c[sl], 1)

    iota = lax.iota(jnp.int32, LANES)
    one = jnp.int32(1)

    for h in range(bpw // ROUND):
        hbase = h * ROUND
        copies = []
        for j in range(ROUND // IDX_CHUNK):
            off = j * IDX_CHUNK
            copies.append(pltpu.async_copy(
                table_in.at[pid_t.at[pl.ds(hbase + off, IDX_CHUNK)]],
                rows_t.at[pl.ds(off, IDX_CHUNK)], sem))
            copies.append(pltpu.async_copy(
                table_out.at[pid_c.at[pl.ds(hbase + off, IDX_CHUNK)]],
                rows_c.at[pl.ds(off, IDX_CHUNK)], sem))
        for cp in copies:
            cp.wait()

        def group(g, carry):
            rbase = g * LANES
            idt16 = ids_t[pl.ds(hbase + rbase, LANES)]
            idc16 = ids_c[pl.ds(hbase + rbase, LANES)]
            for r in range(LANES):
                row = rbase + r
                pt = jnp.bitwise_and(idt16[r], one)
                pc = jnp.bitwise_and(idc16[r], one)
                acc = None
                for c in range(EMBED // LANES):
                    tlo = rows_t[row, pl.ds(c * LANES, LANES)]
                    thi = rows_t[row, pl.ds(EMBED + c * LANES, LANES)]
                    clo = rows_c[row, pl.ds(c * LANES, LANES)]
                    chi = rows_c[row, pl.ds(EMBED + c * LANES, LANES)]
                    tsel = jnp.where(pt == 1, thi, tlo)
                    csel = jnp.where(pc == 1, chi, clo)
                    prod = tsel * csel
                    acc = prod if acc is None else acc + prod
                plsc.store_scatter(tpose, [iota * LANES + r], acc)
            colsum = tpose[pl.ds(0, LANES)]
            for l in range(1, LANES):
                colsum = colsum + tpose[pl.ds(l * LANES, LANES)]
            out_v[pl.ds(hbase + rbase, LANES)] = colsum
            return carry

        lax.fori_loop(0, ROUND // LANES, group, 0)

    pltpu.sync_copy(out_v, o_hbm.at[pl.ds(base, bpw)])


def kernel(target_ids, context_ids, in_embed, out_embed):
    batch = target_ids.shape[0]
    bpw = batch // NWORKERS
    mesh = plsc.VectorSubcoreMesh(core_axis_name="c", subcore_axis_name="s")
    f = pl.kernel(
        functools.partial(_sc_body, bpw),
        out_type=jax.ShapeDtypeStruct((batch,), jnp.float32),
        mesh=mesh,
        scratch_types=[
            pltpu.VMEM((bpw,), jnp.int32),
            pltpu.VMEM((bpw,), jnp.int32),
            pltpu.VMEM((bpw,), jnp.int32),
            pltpu.VMEM((bpw,), jnp.int32),
            pltpu.VMEM((ROUND, PAIR), jnp.float32),
            pltpu.VMEM((ROUND, PAIR), jnp.float32),
            pltpu.VMEM((LANES * LANES,), jnp.float32),
            pltpu.VMEM((bpw,), jnp.float32),
            pltpu.SemaphoreType.DMA,
        ],
        compiler_params=pltpu.CompilerParams(needs_layout_passes=False,
                                             use_tc_tiling_on_sc=True),
    )
    tbl_in = _tc_pair(in_embed)
    tbl_out = _tc_pair(out_embed)
    return f(target_ids.astype(jnp.int32), context_ids.astype(jnp.int32),
             tbl_in, tbl_out)


# speedup vs baseline: 1.6687x; 1.6687x over previous
import functools

import jax
import jax.numpy as jnp
from jax import lax
from jax.experimental import pallas as pl
from jax.experimental.pallas import tpu as pltpu
from jax.experimental.pallas import tpu_sc as plsc

EMBED = 64
LANES = 16
NCORES = 2
NSUB = 16
NWORKERS = NCORES * NSUB
WAVE = 32


def _body(bpw, tid_hbm, cid_hbm, table_in, table_out, o_hbm,
          ids_t, ids_c, buf_t, buf_c, tpose, out_v, sem):
    wid = lax.axis_index("s") * NCORES + lax.axis_index("c")
    base = wid * bpw

    pltpu.sync_copy(tid_hbm.at[pl.ds(base, bpw)], ids_t)
    pltpu.sync_copy(cid_hbm.at[pl.ds(base, bpw)], ids_c)

    iota = lax.iota(jnp.int32, LANES)

    def wave_body(w, carry):
        wbase = w * WAVE
        copies = []
        for g in range(WAVE // LANES):
            idt16 = ids_t[pl.ds(wbase + g * LANES, LANES)]
            idc16 = ids_c[pl.ds(wbase + g * LANES, LANES)]
            for r in range(LANES):
                i = g * LANES + r
                copies.append(pltpu.async_copy(
                    table_in.at[idt16[r]], buf_t.at[i], sem))
                copies.append(pltpu.async_copy(
                    table_out.at[idc16[r]], buf_c.at[i], sem))
        for cp in copies:
            cp.wait()
        for g in range(WAVE // LANES):
            for r in range(LANES):
                i = g * LANES + r
                acc = buf_t[i, pl.ds(0, LANES)] * buf_c[i, pl.ds(0, LANES)]
                for c in range(1, EMBED // LANES):
                    acc = acc + (buf_t[i, pl.ds(c * LANES, LANES)] *
                                 buf_c[i, pl.ds(c * LANES, LANES)])
                plsc.store_scatter(tpose, [iota * LANES + r], acc)
            colsum = tpose[pl.ds(0, LANES)]
            for l in range(1, LANES):
                colsum = colsum + tpose[pl.ds(l * LANES, LANES)]
            out_v[pl.ds(wbase + g * LANES, LANES)] = colsum
        return carry

    lax.fori_loop(0, bpw // WAVE, wave_body, 0)
    pltpu.sync_copy(out_v, o_hbm.at[pl.ds(base, bpw)])


def kernel(target_ids, context_ids, in_embed, out_embed):
    batch = target_ids.shape[0]
    bpw = batch // NWORKERS
    mesh = plsc.VectorSubcoreMesh(core_axis_name="c", subcore_axis_name="s")
    f = pl.kernel(
        functools.partial(_body, bpw),
        out_type=jax.ShapeDtypeStruct((batch,), jnp.float32),
        mesh=mesh,
        scratch_types=[
            pltpu.VMEM((bpw,), jnp.int32),
            pltpu.VMEM((bpw,), jnp.int32),
            pltpu.VMEM((WAVE, EMBED), jnp.float32),
            pltpu.VMEM((WAVE, EMBED), jnp.float32),
            pltpu.VMEM((LANES * LANES,), jnp.float32),
            pltpu.VMEM((bpw,), jnp.float32),
            pltpu.SemaphoreType.DMA,
        ],
        compiler_params=pltpu.CompilerParams(needs_layout_passes=False),
    )
    return f(target_ids.astype(jnp.int32), context_ids.astype(jnp.int32),
             in_embed, out_embed)
